# Initial kernel scaffold; baseline (speedup 1.0000x reference)
import jax
import jax.numpy as jnp
from jax.experimental import pallas as pl


def kernel(x, edge_index, W0, b0, W1, b1, W2, b2):
    raise NotImplementedError("write your pallas kernel here")


# speedup vs baseline: 14.8377x; 14.8377x over previous
import functools

import jax
import jax.numpy as jnp
from jax import lax
from jax.experimental import pallas as pl
from jax.experimental.pallas import tpu as pltpu
from jax.experimental.pallas import tpu_sc as plsc

N = 10000
E = 320000
NGRP = E // 128
NW = 32
KMAX = (NGRP + NW - 1) // NW
ROWS_PER_TILE = N // 16


def _sc_mesh():
    return plsc.VectorSubcoreMesh(core_axis_name="c", subcore_axis_name="s")


_SC_PARAMS = pltpu.CompilerParams(needs_layout_passes=False,
                                  use_tc_tiling_on_sc=False)


@functools.partial(
    pl.kernel,
    out_type=jax.ShapeDtypeStruct((NW, N), jnp.float32),
    mesh=_sc_mesh(),
    scratch_types=[
        pltpu.VMEM((128,), jnp.int32),
        pltpu.VMEM((N,), jnp.float32),
    ],
    compiler_params=_SC_PARAMS,
)
def _deg(dst_hbm, out_hbm, dst_v, deg_v):
    c = lax.axis_index("c")
    s = lax.axis_index("s")
    wid = s * 2 + c

    zeros16 = jnp.zeros((16,), jnp.float32)
    ones16 = jnp.ones((16,), jnp.float32)

    def zbody(i, carry):
        deg_v[pl.ds(i * 16, 16)] = zeros16
        return carry

    lax.fori_loop(0, N // 16, zbody, 0)

    def ebody(k, carry):
        r = wid + NW * k

        @pl.when(r < NGRP)
        def _():
            pltpu.sync_copy(dst_hbm.at[pl.ds(r * 128, 128)], dst_v)
            for j in range(8):
                idx = dst_v[pl.ds(j * 16, 16)]
                plsc.addupdate_scatter(deg_v, [idx], ones16)

        return carry

    lax.fori_loop(0, KMAX, ebody, 0)
    pltpu.sync_copy(deg_v, out_hbm.at[wid])


def _make_prop(W):
    @functools.partial(
        pl.kernel,
        out_type=jax.ShapeDtypeStruct((2, N, W), jnp.float32),
        mesh=_sc_mesh(),
        scratch_types=[
            pltpu.VMEM((128,), jnp.int32),
            pltpu.VMEM((128,), jnp.int32),
            pltpu.VMEM((128, W), jnp.float32),
            pltpu.VMEM((125, W), jnp.float32),
            pltpu.VMEM_SHARED((N, W), jnp.float32),
            pltpu.SemaphoreType.DMA,
        ],
        compiler_params=_SC_PARAMS,
    )
    def _prop(hs_hbm, src_hbm, dst_hbm, out_hbm, src_v, dst_v, rows_v, z_v,
              acc, sem):
        c = lax.axis_index("c")
        s = lax.axis_index("s")
        wid = s * 2 + c

        zeros16 = jnp.zeros((16,), jnp.float32)

        def zbody(i, carry):
            for j in range(W // 16):
                z_v[i, pl.ds(j * 16, 16)] = zeros16
            return carry

        lax.fori_loop(0, 125, zbody, 0)
        for i in range(ROWS_PER_TILE // 125):
            pltpu.sync_copy(z_v, acc.at[pl.ds(s * ROWS_PER_TILE + i * 125, 125), :])
        plsc.subcore_barrier()

        def ebody(k, carry):
            r = wid + NW * k

            @pl.when(r < NGRP)
            def _():
                base = r * 128
                pltpu.sync_copy(src_hbm.at[pl.ds(base, 128)], src_v)
                pltpu.sync_copy(dst_hbm.at[pl.ds(base, 128)], dst_v)
                pltpu.async_copy(hs_hbm.at[src_v], rows_v, sem).wait()
                pltpu.sync_copy(rows_v, acc.at[dst_v], add=True)

            return carry

        lax.fori_loop(0, KMAX, ebody, 0)
        plsc.subcore_barrier()
        pltpu.sync_copy(
            acc.at[pl.ds(s * ROWS_PER_TILE, ROWS_PER_TILE), :],
            out_hbm.at[c, pl.ds(s * ROWS_PER_TILE, ROWS_PER_TILE), :],
        )

    return _prop


_prop128 = _make_prop(128)
_prop64 = _make_prop(64)


BR = 2000
GRID = N // BR


def _gelu(t):
    return 0.5 * t * (1.0 + lax.erf(t * 0.7071067811865476))


def _dinv_of(deg_ref):
    return lax.rsqrt(jnp.sum(deg_ref[...], axis=1) + 2.0)[:, None]


def _tc0_body(deg_ref, x_ref, w_ref, hs_ref):
    dinv = _dinv_of(deg_ref)
    hs_ref[...] = jnp.dot(x_ref[...], w_ref[...],
                          preferred_element_type=jnp.float32) * dinv


def _tc1_body(acc_ref, hs_ref, deg_ref, b_ref, w_ref, h1_ref, hs1_ref):
    dinv = _dinv_of(deg_ref)
    t = (acc_ref[0] + acc_ref[1] + 2.0 * hs_ref[...]) * dinv + b_ref[...]
    h1 = _gelu(t)
    h1_ref[...] = h1
    hs1_ref[...] = jnp.dot(h1, w_ref[...],
                           preferred_element_type=jnp.float32) * dinv


def _tc2_body(acc_ref, hs_ref, h1_ref, deg_ref, b_ref, w_ref, hs2_ref):
    dinv = _dinv_of(deg_ref)
    t = h1_ref[...] + (acc_ref[0] + acc_ref[1] + 2.0 * hs_ref[...]) * dinv \
        + b_ref[...]
    h2 = _gelu(t)
    hs2_ref[...] = jnp.dot(h2, w_ref[...],
                           preferred_element_type=jnp.float32) * dinv


def _tc3_body(acc_ref, hs_ref, deg_ref, b_ref, out_ref):
    dinv = _dinv_of(deg_ref)
    out_ref[...] = (acc_ref[0] + acc_ref[1] + 2.0 * hs_ref[...]) * dinv \
        + b_ref[...]


def _row_spec(w):
    return pl.BlockSpec((BR, w), lambda i: (i, 0))


def _acc_spec(w):
    return pl.BlockSpec((2, BR, w), lambda i: (0, i, 0))


_DEG_SPEC = pl.BlockSpec((BR, NW), lambda i: (i, 0))


def _full_spec(shape):
    return pl.BlockSpec(shape, lambda i: tuple(0 for _ in shape))


def _tc0(deg, x, w0):
    return pl.pallas_call(
        _tc0_body,
        grid=(GRID,),
        in_specs=[_DEG_SPEC, _row_spec(128), _full_spec((128, 128))],
        out_specs=_row_spec(128),
        out_shape=jax.ShapeDtypeStruct((N, 128), jnp.float32),
    )(deg, x, w0)


def _tc1(acc, hs, deg, b, w1):
    return pl.pallas_call(
        _tc1_body,
        grid=(GRID,),
        in_specs=[_acc_spec(128), _row_spec(128), _DEG_SPEC,
                  _full_spec((1, 128)), _full_spec((128, 128))],
        out_specs=[_row_spec(128), _row_spec(128)],
        out_shape=[jax.ShapeDtypeStruct((N, 128), jnp.float32),
                   jax.ShapeDtypeStruct((N, 128), jnp.float32)],
    )(acc, hs, deg, b, w1)


def _tc2(acc, hs, h1, deg, b, w2):
    return pl.pallas_call(
        _tc2_body,
        grid=(GRID,),
        in_specs=[_acc_spec(128), _row_spec(128), _row_spec(128), _DEG_SPEC,
                  _full_spec((1, 128)), _full_spec((128, 64))],
        out_specs=_row_spec(64),
        out_shape=jax.ShapeDtypeStruct((N, 64), jnp.float32),
    )(acc, hs, h1, deg, b, w2)


def _tc3(acc, hs, deg, b):
    return pl.pallas_call(
        _tc3_body,
        grid=(GRID,),
        in_specs=[_acc_spec(64), _row_spec(64), _DEG_SPEC,
                  _full_spec((1, 64))],
        out_specs=_row_spec(64),
        out_shape=jax.ShapeDtypeStruct((N, 64), jnp.float32),
    )(acc, hs, deg, b)


@jax.jit
def kernel(x, edge_index, W0, b0, W1, b1, W2, b2):
    src = edge_index[0]
    dst = edge_index[1]

    deg = _deg(dst).T

    hs0 = _tc0(deg, x, W0)
    acc0 = _prop128(hs0, src, dst)
    h1, hs1 = _tc1(acc0, hs0, deg, b0.reshape(1, 128), W1)
    acc1 = _prop128(hs1, src, dst)
    hs2 = _tc2(acc1, hs1, h1, deg, b1.reshape(1, 128), W2)
    acc2 = _prop64(hs2, src, dst)
    out = _tc3(acc2, hs2, deg, b2.reshape(1, 64))
    return out


# speedup vs baseline: 25.1707x; 1.6964x over previous
import functools

import jax
import jax.numpy as jnp
from jax import lax
from jax.experimental import pallas as pl
from jax.experimental.pallas import tpu as pltpu
from jax.experimental.pallas import tpu_sc as plsc

N = 10000
E = 320000
NGRP = E // 128
NW = 32
KMAX = (NGRP + NW - 1) // NW
ROWS_PER_TILE = N // 16


def _sc_mesh():
    return plsc.VectorSubcoreMesh(core_axis_name="c", subcore_axis_name="s")


_SC_PARAMS = pltpu.CompilerParams(needs_layout_passes=False,
                                  use_tc_tiling_on_sc=False)


@functools.partial(
    pl.kernel,
    out_type=jax.ShapeDtypeStruct((NW, N), jnp.float32),
    mesh=_sc_mesh(),
    scratch_types=[
        pltpu.VMEM((128,), jnp.int32),
        pltpu.VMEM((N,), jnp.float32),
    ],
    compiler_params=_SC_PARAMS,
)
def _deg(dst_hbm, out_hbm, dst_v, deg_v):
    c = lax.axis_index("c")
    s = lax.axis_index("s")
    wid = s * 2 + c

    zeros16 = jnp.zeros((16,), jnp.float32)
    ones16 = jnp.ones((16,), jnp.float32)

    def zbody(i, carry):
        deg_v[pl.ds(i * 16, 16)] = zeros16
        return carry

    lax.fori_loop(0, N // 16, zbody, 0)

    def ebody(k, carry):
        r = wid + NW * k

        @pl.when(r < NGRP)
        def _():
            pltpu.sync_copy(dst_hbm.at[pl.ds(r * 128, 128)], dst_v)
            for j in range(8):
                idx = dst_v[pl.ds(j * 16, 16)]
                plsc.addupdate_scatter(deg_v, [idx], ones16)

        return carry

    lax.fori_loop(0, KMAX, ebody, 0)
    pltpu.sync_copy(deg_v, out_hbm.at[wid])


def _make_prop(W):
    @functools.partial(
        pl.kernel,
        out_type=jax.ShapeDtypeStruct((2, N, W), jnp.float32),
        mesh=_sc_mesh(),
        scratch_types=(
            [pltpu.VMEM((2, 128), jnp.int32)] * 4
            + [pltpu.VMEM((128, W), jnp.float32)] * 2
            + [pltpu.VMEM((125, W), jnp.float32)]
            + [pltpu.VMEM_SHARED((N, W), jnp.float32)]
            + [pltpu.SemaphoreType.DMA] * 8
        ),
        compiler_params=_SC_PARAMS,
    )
    def _prop(hs_hbm, ei_hbm, out_hbm, ev0, ev1, ev2, ev3, rows0, rows1, z_v,
              acc, si0, si1, si2, si3, sg0, sg1, ss0, ss1):
        c = lax.axis_index("c")
        s = lax.axis_index("s")
        wid = s * 2 + c

        evs = (ev0, ev1, ev2, ev3)
        rows = (rows0, rows1)
        sem_i = (si0, si1, si2, si3)
        sem_g = (sg0, sg1)
        sem_s = (ss0, ss1)

        zeros16 = jnp.zeros((16,), jnp.float32)

        def zbody(i, carry):
            for j in range(W // 16):
                z_v[i, pl.ds(j * 16, 16)] = zeros16
            return carry

        lax.fori_loop(0, 125, zbody, 0)
        for i in range(ROWS_PER_TILE // 125):
            pltpu.sync_copy(z_v, acc.at[pl.ds(s * ROWS_PER_TILE + i * 125, 125), :])
        plsc.subcore_barrier()

        def guard(k, u, fn):
            cond = (jnp.int32(k) >= 0) & (wid + NW * k < NGRP)

            @pl.when(cond)
            def _():
                fn(wid + NW * k, u % 2, u % 4)

        def idx_start(r, b, q):
            pltpu.async_copy(ei_hbm.at[r], evs[q], sem_i[q])

        def idx_wait(r, b, q):
            pltpu.make_async_copy(ei_hbm.at[r], evs[q], sem_i[q]).wait()

        def gather_start(r, b, q):
            pltpu.async_copy(hs_hbm.at[evs[q].at[0]], rows[b], sem_g[b])

        def gather_wait(r, b, q):
            pltpu.make_async_copy(hs_hbm.at[evs[q].at[0]], rows[b],
                                  sem_g[b]).wait()

        def scat_start(r, b, q):
            pltpu.async_copy(rows[b], acc.at[evs[q].at[1]], sem_s[b], add=True)

        def scat_wait(r, b, q):
            pltpu.make_async_copy(rows[b], acc.at[evs[q].at[1]],
                                  sem_s[b]).wait()

        def step(k, u):
            guard(k, u, gather_wait)
            guard(k, u, scat_start)
            guard(k + 1, u + 1, idx_wait)
            guard(k - 1, u - 1, scat_wait)
            guard(k + 1, u + 1, gather_start)
            guard(k + 3, u + 3, idx_start)

        for k in range(3):
            guard(k, k, idx_start)
        guard(0, 0, idx_wait)
        guard(0, 0, gather_start)

        def ebody(t, carry):
            for u in range(4):
                step(4 * t + u, u)
            return carry

        nfull = KMAX // 4
        lax.fori_loop(0, nfull, ebody, 0)
        for k in range(4 * nfull, KMAX):
            step(k, k)
        guard(KMAX - 1, KMAX - 1, scat_wait)
        plsc.subcore_barrier()
        pltpu.sync_copy(
            acc.at[pl.ds(s * ROWS_PER_TILE, ROWS_PER_TILE), :],
            out_hbm.at[c, pl.ds(s * ROWS_PER_TILE, ROWS_PER_TILE), :],
        )

    return _prop


_prop128 = _make_prop(128)
_prop64 = _make_prop(64)


BR = 2000
GRID = N // BR


def _gelu(t):
    return 0.5 * t * (1.0 + lax.erf(t * 0.7071067811865476))


def _dinv_of(deg_ref):
    return lax.rsqrt(jnp.sum(deg_ref[...], axis=1) + 2.0)[:, None]


def _tc0_body(deg_ref, x_ref, w_ref, hs_ref):
    dinv = _dinv_of(deg_ref)
    hs_ref[...] = jnp.dot(x_ref[...], w_ref[...],
                          preferred_element_type=jnp.float32) * dinv


def _tc1_body(acc_ref, hs_ref, deg_ref, b_ref, w_ref, h1_ref, hs1_ref):
    dinv = _dinv_of(deg_ref)
    t = (acc_ref[0] + acc_ref[1] + 2.0 * hs_ref[...]) * dinv + b_ref[...]
    h1 = _gelu(t)
    h1_ref[...] = h1
    hs1_ref[...] = jnp.dot(h1, w_ref[...],
                           preferred_element_type=jnp.float32) * dinv


def _tc2_body(acc_ref, hs_ref, h1_ref, deg_ref, b_ref, w_ref, hs2_ref):
    dinv = _dinv_of(deg_ref)
    t = h1_ref[...] + (acc_ref[0] + acc_ref[1] + 2.0 * hs_ref[...]) * dinv \
        + b_ref[...]
    h2 = _gelu(t)
    hs2_ref[...] = jnp.dot(h2, w_ref[...],
                           preferred_element_type=jnp.float32) * dinv


def _tc3_body(acc_ref, hs_ref, deg_ref, b_ref, out_ref):
    dinv = _dinv_of(deg_ref)
    out_ref[...] = (acc_ref[0] + acc_ref[1] + 2.0 * hs_ref[...]) * dinv \
        + b_ref[...]


def _row_spec(w):
    return pl.BlockSpec((BR, w), lambda i: (i, 0))


def _acc_spec(w):
    return pl.BlockSpec((2, BR, w), lambda i: (0, i, 0))


_DEG_SPEC = pl.BlockSpec((BR, NW), lambda i: (i, 0))


def _full_spec(shape):
    return pl.BlockSpec(shape, lambda i: tuple(0 for _ in shape))


def _tc0(deg, x, w0):
    return pl.pallas_call(
        _tc0_body,
        grid=(GRID,),
        in_specs=[_DEG_SPEC, _row_spec(128), _full_spec((128, 128))],
        out_specs=_row_spec(128),
        out_shape=jax.ShapeDtypeStruct((N, 128), jnp.float32),
    )(deg, x, w0)


def _tc1(acc, hs, deg, b, w1):
    return pl.pallas_call(
        _tc1_body,
        grid=(GRID,),
        in_specs=[_acc_spec(128), _row_spec(128), _DEG_SPEC,
                  _full_spec((1, 128)), _full_spec((128, 128))],
        out_specs=[_row_spec(128), _row_spec(128)],
        out_shape=[jax.ShapeDtypeStruct((N, 128), jnp.float32),
                   jax.ShapeDtypeStruct((N, 128), jnp.float32)],
    )(acc, hs, deg, b, w1)


def _tc2(acc, hs, h1, deg, b, w2):
    return pl.pallas_call(
        _tc2_body,
        grid=(GRID,),
        in_specs=[_acc_spec(128), _row_spec(128), _row_spec(128), _DEG_SPEC,
                  _full_spec((1, 128)), _full_spec((128, 64))],
        out_specs=_row_spec(64),
        out_shape=jax.ShapeDtypeStruct((N, 64), jnp.float32),
    )(acc, hs, h1, deg, b, w2)


def _tc3(acc, hs, deg, b):
    return pl.pallas_call(
        _tc3_body,
        grid=(GRID,),
        in_specs=[_acc_spec(64), _row_spec(64), _DEG_SPEC,
                  _full_spec((1, 64))],
        out_specs=_row_spec(64),
        out_shape=jax.ShapeDtypeStruct((N, 64), jnp.float32),
    )(acc, hs, deg, b)


@jax.jit
def kernel(x, edge_index, W0, b0, W1, b1, W2, b2):
    dst = edge_index[1]
    ei = edge_index.reshape(2, NGRP, 128).transpose(1, 0, 2)

    deg = _deg(dst).T

    hs0 = _tc0(deg, x, W0)
    acc0 = _prop128(hs0, ei)
    h1, hs1 = _tc1(acc0, hs0, deg, b0.reshape(1, 128), W1)
    acc1 = _prop128(hs1, ei)
    hs2 = _tc2(acc1, hs1, h1, deg, b1.reshape(1, 128), W2)
    acc2 = _prop64(hs2, ei)
    out = _tc3(acc2, hs2, deg, b2.reshape(1, 64))
    return out
